# baseline (speedup 1.0000x reference)
import functools

import jax
import jax.numpy as jnp
from jax import lax
from jax.experimental import pallas as pl
from jax.experimental.pallas import tpu as pltpu
from jax.experimental.pallas import tpu_sc as plsc

POOL = 4096
LENGTH = 8
DIM = 1024
TOPK = 4
BATCH = 1024
SEQ = 64
KDIM = 2048

PBLK = 512
NPBLK = POOL // PBLK

NCORES = 2
NSUB = 16
NWORK = NCORES * NSUB
BPW = BATCH // NWORK
IPW = BPW * TOPK
HALF = (LENGTH * DIM) // 2


def _sim_topk_body(x_ref, pk_ref, idx_out, w16_out, rs_out, vals_s, idxs_s, xinv_s):
    p = pl.program_id(0)

    @pl.when(p == 0)
    def _init():
        x = x_ref[...]
        xn = jnp.sqrt(jnp.sum(x * x, axis=1, keepdims=True))
        xinv_s[...] = 1.0 / jnp.maximum(xn, 1e-12)
        vals_s[...] = jnp.full((BATCH, TOPK), -jnp.inf, jnp.float32)
        idxs_s[...] = jnp.zeros((BATCH, TOPK), jnp.int32)

    pk = pk_ref[...]
    pkn = pk * (1.0 / jnp.maximum(jnp.sqrt(jnp.sum(pk * pk, axis=1, keepdims=True)), 1e-12))
    sim = lax.dot_general(x_ref[...], pkn, (((1,), (1,)), ((), ())),
                          preferred_element_type=jnp.float32)
    sim = sim * xinv_s[...]

    col = lax.broadcasted_iota(jnp.int32, (BATCH, PBLK), 1)
    bv, bi = [], []
    v = sim
    for _ in range(TOPK):
        m = jnp.max(v, axis=1, keepdims=True)
        ii = jnp.min(jnp.where(v == m, col, PBLK), axis=1, keepdims=True)
        bv.append(m)
        bi.append(ii + p * PBLK)
        v = jnp.where(col == ii, -jnp.inf, v)
    bv = jnp.concatenate(bv, axis=1)
    bi = jnp.concatenate(bi, axis=1)

    cv = jnp.concatenate([vals_s[...], bv], axis=1)
    ci = jnp.concatenate([idxs_s[...], bi], axis=1)
    nv, ni = [], []
    for _ in range(TOPK):
        m = jnp.max(cv, axis=1, keepdims=True)
        ii = jnp.min(jnp.where(cv == m, ci, jnp.int32(2 ** 30)), axis=1, keepdims=True)
        nv.append(m)
        ni.append(ii)
        cv = jnp.where((cv == m) & (ci == ii), -jnp.inf, cv)
    vals = jnp.concatenate(nv, axis=1)
    idxs = jnp.concatenate(ni, axis=1)
    vals_s[...] = vals
    idxs_s[...] = idxs

    @pl.when(p == NPBLK - 1)
    def _fin():
        idx_out[...] = jnp.concatenate([idxs, idxs], axis=1)
        mx = jnp.max(vals, axis=1, keepdims=True)
        e = jnp.exp(vals - mx)
        w = e / jnp.sum(e, axis=1, keepdims=True) / TOPK
        w16_out[...] = jnp.broadcast_to(w[:, :, None], (BATCH, TOPK, 16))
        rs_out[...] = (jnp.sum(vals) / BATCH).reshape(1, 1)


def _sim_topk(x_key, prompt_key):
    return pl.pallas_call(
        _sim_topk_body,
        grid=(NPBLK,),
        in_specs=[
            pl.BlockSpec((BATCH, KDIM), lambda p: (0, 0)),
            pl.BlockSpec((PBLK, KDIM), lambda p: (p, 0)),
        ],
        out_specs=[
            pl.BlockSpec((BATCH, 2 * TOPK), lambda p: (0, 0)),
            pl.BlockSpec((BATCH, TOPK, 16), lambda p: (0, 0, 0)),
            pl.BlockSpec((1, 1), lambda p: (0, 0)),
        ],
        out_shape=[
            jax.ShapeDtypeStruct((BATCH, 2 * TOPK), jnp.int32),
            jax.ShapeDtypeStruct((BATCH, TOPK, 16), jnp.float32),
            jax.ShapeDtypeStruct((1, 1), jnp.float32),
        ],
        scratch_shapes=[
            pltpu.VMEM((BATCH, TOPK), jnp.float32),
            pltpu.VMEM((BATCH, TOPK), jnp.int32),
            pltpu.VMEM((BATCH, 1), jnp.float32),
        ],
        compiler_params=pltpu.CompilerParams(
            dimension_semantics=("arbitrary",),
        ),
    )(x_key, prompt_key)


def _sc_combine_body(tab_ref, idxp_ref, w_ref, out_ref,
                     idxp_v, w_v, buf_a, buf_b, buf_c, sem_a, sem_b, sem_c):
    cid = lax.axis_index("c")
    sid = lax.axis_index("s")
    wid = sid * NCORES + cid
    base_i = wid * IPW

    pltpu.sync_copy(idxp_ref.at[pl.ds(base_i * 2, IPW * 2)], idxp_v)
    pltpu.sync_copy(w_ref.at[pl.ds(base_i * 16, IPW * 16)], w_v)

    bufs = (buf_a, buf_b, buf_c)
    sems = (sem_a, sem_b, sem_c)

    def start(j, buf, sem):
        pltpu.async_copy(tab_ref.at[idxp_v.at[pl.ds(j * 8, TOPK)]], buf, sem)

    def drain(buf, sem):
        pltpu.make_async_copy(tab_ref.at[pl.ds(0, TOPK)], buf, sem).wait()

    def combine(j, buf):
        w0 = w_v[pl.ds((j * TOPK + 0) * 16, 16)]
        w1 = w_v[pl.ds((j * TOPK + 1) * 16, 16)]
        w2 = w_v[pl.ds((j * TOPK + 2) * 16, 16)]
        w3 = w_v[pl.ds((j * TOPK + 3) * 16, 16)]

        def chunk(ci2, _):
            for u in range(2):
                off = (ci2 * 2 + u) * 16
                for l in range(LENGTH):
                    acc = w0 * buf[0, l, pl.ds(off, 16)]
                    acc = acc + w1 * buf[1, l, pl.ds(off, 16)]
                    acc = acc + w2 * buf[2, l, pl.ds(off, 16)]
                    acc = acc + w3 * buf[3, l, pl.ds(off, 16)]
                    buf[0, l, pl.ds(off, 16)] = acc
            return 0
        lax.fori_loop(0, DIM // 32, chunk, 0)
        pltpu.sync_copy(buf.at[0], out_ref.at[wid * BPW + j])

    for s in range(3):
        start(s, bufs[s], sems[s])

    def ring(t, _):
        for s in range(3):
            j = 3 * t + s

            @pl.when(j < BPW)
            def _():
                drain(bufs[s], sems[s])
                combine(j, bufs[s])

            @pl.when(j + 3 < BPW)
            def _():
                start(j + 3, bufs[s], sems[s])
        return 0

    lax.fori_loop(0, (BPW + 2) // 3, ring, 0)


def _sc_combine(prompt, idx_pad, w_flat):
    mesh = plsc.VectorSubcoreMesh(core_axis_name="c", subcore_axis_name="s")
    fn = functools.partial(
        pl.kernel,
        out_type=jax.ShapeDtypeStruct((BATCH, LENGTH, DIM), jnp.float32),
        mesh=mesh,
        scratch_types=[
            pltpu.VMEM((2 * IPW,), jnp.int32),
            pltpu.VMEM((IPW * 16,), jnp.float32),
            pltpu.VMEM((TOPK, LENGTH, DIM), jnp.float32),
            pltpu.VMEM((TOPK, LENGTH, DIM), jnp.float32),
            pltpu.VMEM((TOPK, LENGTH, DIM), jnp.float32),
            pltpu.SemaphoreType.DMA,
            pltpu.SemaphoreType.DMA,
            pltpu.SemaphoreType.DMA,
        ],
        compiler_params=pltpu.CompilerParams(use_tc_tiling_on_sc=True),
    )(_sc_combine_body)
    return fn(prompt, idx_pad, w_flat)


SHELL_BB = 128


NDMA = 8


def _shell_body(x_ref, out_ref, sem):
    bb = BATCH // NDMA
    for i in range(NDMA):
        pltpu.make_async_copy(
            x_ref.at[pl.ds(i * bb, bb)],
            out_ref.at[pl.ds(i * bb, bb), pl.ds(LENGTH, SEQ)],
            sem,
        ).start()
    for i in range(NDMA):
        pltpu.make_async_copy(
            x_ref.at[pl.ds(i * bb, bb)],
            out_ref.at[pl.ds(i * bb, bb), pl.ds(LENGTH, SEQ)],
            sem,
        ).wait()


def _shell(x_embed):
    return pl.pallas_call(
        _shell_body,
        in_specs=[pl.BlockSpec(memory_space=pltpu.MemorySpace.HBM)],
        out_specs=pl.BlockSpec(memory_space=pltpu.MemorySpace.HBM),
        out_shape=jax.ShapeDtypeStruct((BATCH, LENGTH + SEQ, DIM), jnp.float32),
        scratch_shapes=[pltpu.SemaphoreType.DMA],
    )(x_embed)


def _assemble_body(mean_ref, shell_ref, out_ref):
    del shell_ref
    out_ref[...] = mean_ref[...]


def _assemble(mean3, shell):
    return pl.pallas_call(
        _assemble_body,
        grid=(BATCH // SHELL_BB,),
        in_specs=[
            pl.BlockSpec((SHELL_BB, LENGTH, DIM), lambda b: (b, 0, 0)),
            pl.BlockSpec(memory_space=pltpu.MemorySpace.HBM),
        ],
        out_specs=pl.BlockSpec((SHELL_BB, LENGTH, DIM), lambda b: (b, 0, 0)),
        out_shape=jax.ShapeDtypeStruct((BATCH, LENGTH + SEQ, DIM), jnp.float32),
        input_output_aliases={1: 0},
        compiler_params=pltpu.CompilerParams(
            dimension_semantics=("arbitrary",),
        ),
    )(mean3, shell)


def kernel(x_embed, x_key, prompt, prompt_key):
    shell = _shell(x_embed)
    idxp, w16, rs = _sim_topk(x_key, prompt_key)
    mean = _sc_combine(prompt, idxp.reshape(BATCH * 2 * TOPK), w16.reshape(BATCH * TOPK * 16))
    out = _assemble(mean, shell)
    return rs[0, 0], out


# speedup vs baseline: 23.4875x; 23.4875x over previous
import functools

import jax
import jax.numpy as jnp
from jax import lax
from jax.experimental import pallas as pl
from jax.experimental.pallas import tpu as pltpu
from jax.experimental.pallas import tpu_sc as plsc

POOL = 4096
LENGTH = 8
DIM = 1024
TOPK = 4
BATCH = 1024
SEQ = 64
KDIM = 2048

PBLK = 512
NPBLK = POOL // PBLK

NCORES = 2
NSUB = 16
NWORK = NCORES * NSUB
BPW = BATCH // NWORK
IPW = BPW * TOPK
HALF = (LENGTH * DIM) // 2


def _sim_topk_body(x_ref, pk_ref, idx_out, w16_out, rs_out, vals_s, idxs_s, xinv_s):
    p = pl.program_id(0)

    @pl.when(p == 0)
    def _init():
        x = x_ref[...]
        xn = jnp.sqrt(jnp.sum(x * x, axis=1, keepdims=True))
        xinv_s[...] = 1.0 / jnp.maximum(xn, 1e-12)
        vals_s[...] = jnp.full((BATCH, TOPK), -jnp.inf, jnp.float32)
        idxs_s[...] = jnp.zeros((BATCH, TOPK), jnp.int32)

    pk = pk_ref[...]
    pkn = pk * (1.0 / jnp.maximum(jnp.sqrt(jnp.sum(pk * pk, axis=1, keepdims=True)), 1e-12))
    sim = lax.dot_general(x_ref[...], pkn, (((1,), (1,)), ((), ())),
                          preferred_element_type=jnp.float32)
    sim = sim * xinv_s[...]

    col = lax.broadcasted_iota(jnp.int32, (BATCH, PBLK), 1)
    bv, bi = [], []
    v = sim
    for _ in range(TOPK):
        m = jnp.max(v, axis=1, keepdims=True)
        ii = jnp.min(jnp.where(v == m, col, PBLK), axis=1, keepdims=True)
        bv.append(m)
        bi.append(ii + p * PBLK)
        v = jnp.where(col == ii, -jnp.inf, v)
    bv = jnp.concatenate(bv, axis=1)
    bi = jnp.concatenate(bi, axis=1)

    cv = jnp.concatenate([vals_s[...], bv], axis=1)
    ci = jnp.concatenate([idxs_s[...], bi], axis=1)
    nv, ni = [], []
    for _ in range(TOPK):
        m = jnp.max(cv, axis=1, keepdims=True)
        ii = jnp.min(jnp.where(cv == m, ci, jnp.int32(2 ** 30)), axis=1, keepdims=True)
        nv.append(m)
        ni.append(ii)
        cv = jnp.where((cv == m) & (ci == ii), -jnp.inf, cv)
    vals = jnp.concatenate(nv, axis=1)
    idxs = jnp.concatenate(ni, axis=1)
    vals_s[...] = vals
    idxs_s[...] = idxs

    @pl.when(p == NPBLK - 1)
    def _fin():
        idx_out[...] = jnp.concatenate([idxs, idxs], axis=1)
        mx = jnp.max(vals, axis=1, keepdims=True)
        e = jnp.exp(vals - mx)
        w = e / jnp.sum(e, axis=1, keepdims=True) / TOPK
        w16_out[...] = jnp.broadcast_to(w[:, :, None], (BATCH, TOPK, 16))
        rs_out[...] = (jnp.sum(vals) / BATCH).reshape(1, 1)


def _sim_topk(x_key, prompt_key):
    return pl.pallas_call(
        _sim_topk_body,
        grid=(NPBLK,),
        in_specs=[
            pl.BlockSpec((BATCH, KDIM), lambda p: (0, 0)),
            pl.BlockSpec((PBLK, KDIM), lambda p: (p, 0)),
        ],
        out_specs=[
            pl.BlockSpec((BATCH, 2 * TOPK), lambda p: (0, 0)),
            pl.BlockSpec((BATCH, TOPK, 16), lambda p: (0, 0, 0)),
            pl.BlockSpec((1, 1), lambda p: (0, 0)),
        ],
        out_shape=[
            jax.ShapeDtypeStruct((BATCH, 2 * TOPK), jnp.int32),
            jax.ShapeDtypeStruct((BATCH, TOPK, 16), jnp.float32),
            jax.ShapeDtypeStruct((1, 1), jnp.float32),
        ],
        scratch_shapes=[
            pltpu.VMEM((BATCH, TOPK), jnp.float32),
            pltpu.VMEM((BATCH, TOPK), jnp.int32),
            pltpu.VMEM((BATCH, 1), jnp.float32),
        ],
        compiler_params=pltpu.CompilerParams(
            dimension_semantics=("arbitrary",),
        ),
    )(x_key, prompt_key)


def _sc_combine_body(tab_ref, idxp_ref, w_ref, out_ref,
                     idxp_v, w_v, buf_a, buf_b, buf_c, sem_a, sem_b, sem_c):
    cid = lax.axis_index("c")
    sid = lax.axis_index("s")
    wid = sid * NCORES + cid
    base_i = wid * IPW

    pltpu.sync_copy(idxp_ref.at[pl.ds(base_i * 2, IPW * 2)], idxp_v)
    pltpu.sync_copy(w_ref.at[pl.ds(base_i * 16, IPW * 16)], w_v)

    bufs = (buf_a, buf_b, buf_c)
    sems = (sem_a, sem_b, sem_c)

    def start(j, buf, sem):
        pltpu.async_copy(tab_ref.at[idxp_v.at[pl.ds(j * 8, TOPK)]], buf, sem)

    def drain(buf, sem):
        pltpu.make_async_copy(tab_ref.at[pl.ds(0, TOPK)], buf, sem).wait()

    def combine(j, buf):
        w0 = w_v[pl.ds((j * TOPK + 0) * 16, 16)]
        w1 = w_v[pl.ds((j * TOPK + 1) * 16, 16)]
        w2 = w_v[pl.ds((j * TOPK + 2) * 16, 16)]
        w3 = w_v[pl.ds((j * TOPK + 3) * 16, 16)]

        def chunk(ci2, _):
            for u in range(2):
                off = (ci2 * 2 + u) * 16
                for l in range(LENGTH):
                    acc = w0 * buf[0, l, pl.ds(off, 16)]
                    acc = acc + w1 * buf[1, l, pl.ds(off, 16)]
                    acc = acc + w2 * buf[2, l, pl.ds(off, 16)]
                    acc = acc + w3 * buf[3, l, pl.ds(off, 16)]
                    buf[0, l, pl.ds(off, 16)] = acc
            return 0
        lax.fori_loop(0, DIM // 32, chunk, 0)
        pltpu.sync_copy(buf.at[0], out_ref.at[wid * BPW + j])

    for s in range(3):
        start(s, bufs[s], sems[s])

    def ring(t, _):
        for s in range(3):
            j = 3 * t + s

            @pl.when(j < BPW)
            def _():
                drain(bufs[s], sems[s])
                combine(j, bufs[s])

            @pl.when(j + 3 < BPW)
            def _():
                start(j + 3, bufs[s], sems[s])
        return 0

    lax.fori_loop(0, (BPW + 2) // 3, ring, 0)


def _sc_combine(prompt, idx_pad, w_flat):
    mesh = plsc.VectorSubcoreMesh(core_axis_name="c", subcore_axis_name="s")
    fn = functools.partial(
        pl.kernel,
        out_type=jax.ShapeDtypeStruct((BATCH, LENGTH, DIM), jnp.float32),
        mesh=mesh,
        scratch_types=[
            pltpu.VMEM((2 * IPW,), jnp.int32),
            pltpu.VMEM((IPW * 16,), jnp.float32),
            pltpu.VMEM((TOPK, LENGTH, DIM), jnp.float32),
            pltpu.VMEM((TOPK, LENGTH, DIM), jnp.float32),
            pltpu.VMEM((TOPK, LENGTH, DIM), jnp.float32),
            pltpu.SemaphoreType.DMA,
            pltpu.SemaphoreType.DMA,
            pltpu.SemaphoreType.DMA,
        ],
        compiler_params=pltpu.CompilerParams(use_tc_tiling_on_sc=True),
    )(_sc_combine_body)
    return fn(prompt, idx_pad, w_flat)


SHELL_BB = 128


def _shell_body(x_ref, out_ref):
    out_ref[...] = x_ref[...]


def _shell(x_embed):
    return pl.pallas_call(
        _shell_body,
        grid=(BATCH // SHELL_BB, SEQ // LENGTH),
        in_specs=[pl.BlockSpec((SHELL_BB, LENGTH, DIM), lambda b, j: (b, j, 0))],
        out_specs=pl.BlockSpec((SHELL_BB, LENGTH, DIM), lambda b, j: (b, j + 1, 0)),
        out_shape=jax.ShapeDtypeStruct((BATCH, LENGTH + SEQ, DIM), jnp.float32),
        compiler_params=pltpu.CompilerParams(
            dimension_semantics=("parallel", "arbitrary"),
        ),
    )(x_embed)


def _assemble_body(mean_ref, shell_ref, out_ref):
    del shell_ref
    out_ref[...] = mean_ref[...]


def _assemble(mean3, shell):
    return pl.pallas_call(
        _assemble_body,
        grid=(BATCH // SHELL_BB,),
        in_specs=[
            pl.BlockSpec((SHELL_BB, LENGTH, DIM), lambda b: (b, 0, 0)),
            pl.BlockSpec(memory_space=pltpu.MemorySpace.HBM),
        ],
        out_specs=pl.BlockSpec((SHELL_BB, LENGTH, DIM), lambda b: (b, 0, 0)),
        out_shape=jax.ShapeDtypeStruct((BATCH, LENGTH + SEQ, DIM), jnp.float32),
        input_output_aliases={1: 0},
        compiler_params=pltpu.CompilerParams(
            dimension_semantics=("arbitrary",),
        ),
    )(mean3, shell)


def kernel(x_embed, x_key, prompt, prompt_key):
    shell = _shell(x_embed)
    idxp, w16, rs = _sim_topk(x_key, prompt_key)
    mean = _sc_combine(prompt, idxp.reshape(BATCH * 2 * TOPK), w16.reshape(BATCH * TOPK * 16))
    out = _assemble(mean, shell)
    return rs[0, 0], out


# speedup vs baseline: 23.6555x; 1.0072x over previous
import functools

import jax
import jax.numpy as jnp
from jax import lax
from jax.experimental import pallas as pl
from jax.experimental.pallas import tpu as pltpu
from jax.experimental.pallas import tpu_sc as plsc

POOL = 4096
LENGTH = 8
DIM = 1024
TOPK = 4
BATCH = 1024
SEQ = 64
KDIM = 2048

PBLK = 512
NPBLK = POOL // PBLK

NCORES = 2
NSUB = 16
NWORK = NCORES * NSUB
BPW = BATCH // NWORK
IPW = BPW * TOPK
HALF = (LENGTH * DIM) // 2


def _sim_topk_body(x_ref, pk_ref, idx_out, w16_out, rs_out, vals_s, idxs_s, xinv_s):
    p = pl.program_id(0)

    @pl.when(p == 0)
    def _init():
        x = x_ref[...]
        xn = jnp.sqrt(jnp.sum(x * x, axis=1, keepdims=True))
        xinv_s[...] = 1.0 / jnp.maximum(xn, 1e-12)
        vals_s[...] = jnp.full((BATCH, TOPK), -jnp.inf, jnp.float32)
        idxs_s[...] = jnp.zeros((BATCH, TOPK), jnp.int32)

    pk = pk_ref[...]
    pkn = pk * (1.0 / jnp.maximum(jnp.sqrt(jnp.sum(pk * pk, axis=1, keepdims=True)), 1e-12))
    sim = lax.dot_general(x_ref[...], pkn, (((1,), (1,)), ((), ())),
                          preferred_element_type=jnp.float32)
    sim = sim * xinv_s[...]

    col = lax.broadcasted_iota(jnp.int32, (BATCH, PBLK), 1)
    bv, bi = [], []
    v = sim
    for _ in range(TOPK):
        m = jnp.max(v, axis=1, keepdims=True)
        ii = jnp.min(jnp.where(v == m, col, PBLK), axis=1, keepdims=True)
        bv.append(m)
        bi.append(ii + p * PBLK)
        v = jnp.where(col == ii, -jnp.inf, v)
    bv = jnp.concatenate(bv, axis=1)
    bi = jnp.concatenate(bi, axis=1)

    cv = jnp.concatenate([vals_s[...], bv], axis=1)
    ci = jnp.concatenate([idxs_s[...], bi], axis=1)
    nv, ni = [], []
    for _ in range(TOPK):
        m = jnp.max(cv, axis=1, keepdims=True)
        ii = jnp.min(jnp.where(cv == m, ci, jnp.int32(2 ** 30)), axis=1, keepdims=True)
        nv.append(m)
        ni.append(ii)
        cv = jnp.where((cv == m) & (ci == ii), -jnp.inf, cv)
    vals = jnp.concatenate(nv, axis=1)
    idxs = jnp.concatenate(ni, axis=1)
    vals_s[...] = vals
    idxs_s[...] = idxs

    @pl.when(p == NPBLK - 1)
    def _fin():
        idx_out[...] = jnp.concatenate([idxs, idxs], axis=1)
        mx = jnp.max(vals, axis=1, keepdims=True)
        e = jnp.exp(vals - mx)
        w = e / jnp.sum(e, axis=1, keepdims=True) / TOPK
        w16_out[...] = jnp.broadcast_to(w[:, :, None], (BATCH, TOPK, 16))
        rs_out[...] = (jnp.sum(vals) / BATCH).reshape(1, 1)


def _sim_topk(x_key, prompt_key):
    return pl.pallas_call(
        _sim_topk_body,
        grid=(NPBLK,),
        in_specs=[
            pl.BlockSpec((BATCH, KDIM), lambda p: (0, 0)),
            pl.BlockSpec((PBLK, KDIM), lambda p: (p, 0)),
        ],
        out_specs=[
            pl.BlockSpec((BATCH, 2 * TOPK), lambda p: (0, 0)),
            pl.BlockSpec((BATCH, TOPK, 16), lambda p: (0, 0, 0)),
            pl.BlockSpec((1, 1), lambda p: (0, 0)),
        ],
        out_shape=[
            jax.ShapeDtypeStruct((BATCH, 2 * TOPK), jnp.int32),
            jax.ShapeDtypeStruct((BATCH, TOPK, 16), jnp.float32),
            jax.ShapeDtypeStruct((1, 1), jnp.float32),
        ],
        scratch_shapes=[
            pltpu.VMEM((BATCH, TOPK), jnp.float32),
            pltpu.VMEM((BATCH, TOPK), jnp.int32),
            pltpu.VMEM((BATCH, 1), jnp.float32),
        ],
        compiler_params=pltpu.CompilerParams(
            dimension_semantics=("arbitrary",),
        ),
    )(x_key, prompt_key)


def _sc_combine_body(tab_ref, idxp_ref, w_ref, out_ref,
                     idxp_v, w_v, buf_a, buf_b, buf_c, sem_a, sem_b, sem_c):
    cid = lax.axis_index("c")
    sid = lax.axis_index("s")
    wid = sid * NCORES + cid
    base_i = wid * IPW

    pltpu.sync_copy(idxp_ref.at[pl.ds(base_i * 2, IPW * 2)], idxp_v)
    pltpu.sync_copy(w_ref.at[pl.ds(base_i * 16, IPW * 16)], w_v)

    bufs = (buf_a, buf_b, buf_c)
    sems = (sem_a, sem_b, sem_c)

    def start(j, buf, sem):
        pltpu.async_copy(tab_ref.at[idxp_v.at[pl.ds(j * 8, TOPK)]], buf, sem)

    def drain(buf, sem):
        pltpu.make_async_copy(tab_ref.at[pl.ds(0, TOPK)], buf, sem).wait()

    def combine(j, buf):
        w0 = w_v[pl.ds((j * TOPK + 0) * 16, 16)]
        w1 = w_v[pl.ds((j * TOPK + 1) * 16, 16)]
        w2 = w_v[pl.ds((j * TOPK + 2) * 16, 16)]
        w3 = w_v[pl.ds((j * TOPK + 3) * 16, 16)]

        def chunk(ci2, _):
            for u in range(2):
                off = (ci2 * 2 + u) * 16
                for l in range(LENGTH):
                    acc = w0 * buf[0, l, pl.ds(off, 16)]
                    acc = acc + w1 * buf[1, l, pl.ds(off, 16)]
                    acc = acc + w2 * buf[2, l, pl.ds(off, 16)]
                    acc = acc + w3 * buf[3, l, pl.ds(off, 16)]
                    buf[0, l, pl.ds(off, 16)] = acc
            return 0
        lax.fori_loop(0, DIM // 32, chunk, 0)
        pltpu.sync_copy(buf.at[0], out_ref.at[wid * BPW + j])

    for s in range(3):
        start(s, bufs[s], sems[s])

    def ring(t, _):
        for s in range(3):
            j = 3 * t + s

            @pl.when(j < BPW)
            def _():
                drain(bufs[s], sems[s])
                combine(j, bufs[s])

            @pl.when(j + 3 < BPW)
            def _():
                start(j + 3, bufs[s], sems[s])
        return 0

    lax.fori_loop(0, (BPW + 2) // 3, ring, 0)


def _sc_combine(prompt, idx_pad, w_flat):
    mesh = plsc.VectorSubcoreMesh(core_axis_name="c", subcore_axis_name="s")
    fn = functools.partial(
        pl.kernel,
        out_type=jax.ShapeDtypeStruct((BATCH, LENGTH, DIM), jnp.float32),
        mesh=mesh,
        scratch_types=[
            pltpu.VMEM((2 * IPW,), jnp.int32),
            pltpu.VMEM((IPW * 16,), jnp.float32),
            pltpu.VMEM((TOPK, LENGTH, DIM), jnp.float32),
            pltpu.VMEM((TOPK, LENGTH, DIM), jnp.float32),
            pltpu.VMEM((TOPK, LENGTH, DIM), jnp.float32),
            pltpu.SemaphoreType.DMA,
            pltpu.SemaphoreType.DMA,
            pltpu.SemaphoreType.DMA,
        ],
        compiler_params=pltpu.CompilerParams(use_tc_tiling_on_sc=True),
    )(_sc_combine_body)
    return fn(prompt, idx_pad, w_flat)


SHELL_BB = 256
ASM_BB = 128


def _shell_body(x_ref, out_ref):
    out_ref[...] = x_ref[...]


def _shell(x_embed):
    return pl.pallas_call(
        _shell_body,
        grid=(BATCH // SHELL_BB, SEQ // LENGTH),
        in_specs=[pl.BlockSpec((SHELL_BB, LENGTH, DIM), lambda b, j: (b, j, 0))],
        out_specs=pl.BlockSpec((SHELL_BB, LENGTH, DIM), lambda b, j: (b, j + 1, 0)),
        out_shape=jax.ShapeDtypeStruct((BATCH, LENGTH + SEQ, DIM), jnp.float32),
        compiler_params=pltpu.CompilerParams(
            dimension_semantics=("parallel", "arbitrary"),
        ),
    )(x_embed)


def _assemble_body(mean_ref, shell_ref, out_ref):
    del shell_ref
    out_ref[...] = mean_ref[...]


def _assemble(mean3, shell):
    return pl.pallas_call(
        _assemble_body,
        grid=(BATCH // ASM_BB,),
        in_specs=[
            pl.BlockSpec((ASM_BB, LENGTH, DIM), lambda b: (b, 0, 0)),
            pl.BlockSpec(memory_space=pltpu.MemorySpace.HBM),
        ],
        out_specs=pl.BlockSpec((ASM_BB, LENGTH, DIM), lambda b: (b, 0, 0)),
        out_shape=jax.ShapeDtypeStruct((BATCH, LENGTH + SEQ, DIM), jnp.float32),
        input_output_aliases={1: 0},
        compiler_params=pltpu.CompilerParams(
            dimension_semantics=("arbitrary",),
        ),
    )(mean3, shell)


def kernel(x_embed, x_key, prompt, prompt_key):
    shell = _shell(x_embed)
    idxp, w16, rs = _sim_topk(x_key, prompt_key)
    mean = _sc_combine(prompt, idxp.reshape(BATCH * 2 * TOPK), w16.reshape(BATCH * TOPK * 16))
    out = _assemble(mean, shell)
    return rs[0, 0], out


# speedup vs baseline: 24.6545x; 1.0422x over previous
import functools

import jax
import jax.numpy as jnp
from jax import lax
from jax.experimental import pallas as pl
from jax.experimental.pallas import tpu as pltpu
from jax.experimental.pallas import tpu_sc as plsc

POOL = 4096
LENGTH = 8
DIM = 1024
TOPK = 4
BATCH = 1024
SEQ = 64
KDIM = 2048

PBLK = 512
NPBLK = POOL // PBLK

NCORES = 2
NSUB = 16
NWORK = NCORES * NSUB
BPW = BATCH // NWORK
IPW = BPW * TOPK
HALF = (LENGTH * DIM) // 2


def _sim_topk_body(x_ref, pk_ref, idx_out, w16_out, rs_out, vals_s, idxs_s, xinv_s):
    p = pl.program_id(0)

    @pl.when(p == 0)
    def _init():
        x = x_ref[...]
        xn = jnp.sqrt(jnp.sum(x * x, axis=1, keepdims=True))
        xinv_s[...] = 1.0 / jnp.maximum(xn, 1e-12)
        vals_s[...] = jnp.full((BATCH, TOPK), -jnp.inf, jnp.float32)
        idxs_s[...] = jnp.zeros((BATCH, TOPK), jnp.float32)

    pk = pk_ref[...]
    pkn = pk * (1.0 / jnp.maximum(jnp.sqrt(jnp.sum(pk * pk, axis=1, keepdims=True)), 1e-12))
    sim = lax.dot_general(x_ref[...], pkn, (((1,), (1,)), ((), ())),
                          preferred_element_type=jnp.float32)
    sim = sim * xinv_s[...]

    col = lax.broadcasted_iota(jnp.int32, (BATCH, PBLK), 1).astype(jnp.float32)
    bv, bi = [], []
    v = sim
    for _ in range(TOPK):
        m = jnp.max(v, axis=1, keepdims=True)
        ii = jnp.min(jnp.where(v == m, col, jnp.float32(PBLK)), axis=1, keepdims=True)
        bv.append(m)
        bi.append(ii + jnp.float32(p * PBLK))
        v = jnp.where(col == ii, -jnp.inf, v)
    bv = jnp.concatenate(bv, axis=1)
    bi = jnp.concatenate(bi, axis=1)

    cv = jnp.concatenate([vals_s[...], bv], axis=1)
    ci = jnp.concatenate([idxs_s[...], bi], axis=1)
    nv, ni = [], []
    for _ in range(TOPK):
        m = jnp.max(cv, axis=1, keepdims=True)
        ii = jnp.min(jnp.where(cv == m, ci, jnp.float32(2 ** 24)), axis=1, keepdims=True)
        nv.append(m)
        ni.append(ii)
        cv = jnp.where((cv == m) & (ci == ii), -jnp.inf, cv)
    vals = jnp.concatenate(nv, axis=1)
    idxs = jnp.concatenate(ni, axis=1)
    vals_s[...] = vals
    idxs_s[...] = idxs

    @pl.when(p == NPBLK - 1)
    def _fin():
        idxi = idxs.astype(jnp.int32)
        idx_out[...] = jnp.concatenate([idxi, idxi], axis=1)
        mx = jnp.max(vals, axis=1, keepdims=True)
        e = jnp.exp(vals - mx)
        w = e / jnp.sum(e, axis=1, keepdims=True) / TOPK
        w16_out[...] = jnp.broadcast_to(w[:, :, None], (BATCH, TOPK, 16))
        rs_out[...] = (jnp.sum(vals) / BATCH).reshape(1, 1)


def _sim_topk(x_key, prompt_key):
    return pl.pallas_call(
        _sim_topk_body,
        grid=(NPBLK,),
        in_specs=[
            pl.BlockSpec((BATCH, KDIM), lambda p: (0, 0)),
            pl.BlockSpec((PBLK, KDIM), lambda p: (p, 0)),
        ],
        out_specs=[
            pl.BlockSpec((BATCH, 2 * TOPK), lambda p: (0, 0)),
            pl.BlockSpec((BATCH, TOPK, 16), lambda p: (0, 0, 0)),
            pl.BlockSpec((1, 1), lambda p: (0, 0)),
        ],
        out_shape=[
            jax.ShapeDtypeStruct((BATCH, 2 * TOPK), jnp.int32),
            jax.ShapeDtypeStruct((BATCH, TOPK, 16), jnp.float32),
            jax.ShapeDtypeStruct((1, 1), jnp.float32),
        ],
        scratch_shapes=[
            pltpu.VMEM((BATCH, TOPK), jnp.float32),
            pltpu.VMEM((BATCH, TOPK), jnp.float32),
            pltpu.VMEM((BATCH, 1), jnp.float32),
        ],
        compiler_params=pltpu.CompilerParams(
            dimension_semantics=("arbitrary",),
        ),
    )(x_key, prompt_key)


def _sc_combine_body(tab_ref, idxp_ref, w_ref, out_ref,
                     idxp_v, w_v, buf_a, buf_b, buf_c, sem_a, sem_b, sem_c):
    cid = lax.axis_index("c")
    sid = lax.axis_index("s")
    wid = sid * NCORES + cid
    base_i = wid * IPW

    pltpu.sync_copy(idxp_ref.at[pl.ds(base_i * 2, IPW * 2)], idxp_v)
    pltpu.sync_copy(w_ref.at[pl.ds(base_i * 16, IPW * 16)], w_v)

    bufs = (buf_a, buf_b, buf_c)
    sems = (sem_a, sem_b, sem_c)

    def start(j, buf, sem):
        pltpu.async_copy(tab_ref.at[idxp_v.at[pl.ds(j * 8, TOPK)]], buf, sem)

    def drain(buf, sem):
        pltpu.make_async_copy(tab_ref.at[pl.ds(0, TOPK)], buf, sem).wait()

    def combine(j, buf):
        w0 = w_v[pl.ds((j * TOPK + 0) * 16, 16)]
        w1 = w_v[pl.ds((j * TOPK + 1) * 16, 16)]
        w2 = w_v[pl.ds((j * TOPK + 2) * 16, 16)]
        w3 = w_v[pl.ds((j * TOPK + 3) * 16, 16)]

        def chunk(ci2, _):
            for u in range(2):
                off = (ci2 * 2 + u) * 16
                for l in range(LENGTH):
                    acc = w0 * buf[0, l, pl.ds(off, 16)]
                    acc = acc + w1 * buf[1, l, pl.ds(off, 16)]
                    acc = acc + w2 * buf[2, l, pl.ds(off, 16)]
                    acc = acc + w3 * buf[3, l, pl.ds(off, 16)]
                    buf[0, l, pl.ds(off, 16)] = acc
            return 0
        lax.fori_loop(0, DIM // 32, chunk, 0)
        pltpu.sync_copy(buf.at[0], out_ref.at[wid * BPW + j])

    for s in range(3):
        start(s, bufs[s], sems[s])

    def ring(t, _):
        for s in range(3):
            j = 3 * t + s

            @pl.when(j < BPW)
            def _():
                drain(bufs[s], sems[s])
                combine(j, bufs[s])

            @pl.when(j + 3 < BPW)
            def _():
                start(j + 3, bufs[s], sems[s])
        return 0

    lax.fori_loop(0, (BPW + 2) // 3, ring, 0)


def _sc_combine(prompt, idx_pad, w_flat):
    mesh = plsc.VectorSubcoreMesh(core_axis_name="c", subcore_axis_name="s")
    fn = functools.partial(
        pl.kernel,
        out_type=jax.ShapeDtypeStruct((BATCH, LENGTH, DIM), jnp.float32),
        mesh=mesh,
        scratch_types=[
            pltpu.VMEM((2 * IPW,), jnp.int32),
            pltpu.VMEM((IPW * 16,), jnp.float32),
            pltpu.VMEM((TOPK, LENGTH, DIM), jnp.float32),
            pltpu.VMEM((TOPK, LENGTH, DIM), jnp.float32),
            pltpu.VMEM((TOPK, LENGTH, DIM), jnp.float32),
            pltpu.SemaphoreType.DMA,
            pltpu.SemaphoreType.DMA,
            pltpu.SemaphoreType.DMA,
        ],
        compiler_params=pltpu.CompilerParams(use_tc_tiling_on_sc=True),
    )(_sc_combine_body)
    return fn(prompt, idx_pad, w_flat)


SHELL_BB = 256
ASM_BB = 128


def _shell_body(x_ref, out_ref):
    out_ref[...] = x_ref[...]


def _shell(x_embed):
    return pl.pallas_call(
        _shell_body,
        grid=(BATCH // SHELL_BB, SEQ // LENGTH),
        in_specs=[pl.BlockSpec((SHELL_BB, LENGTH, DIM), lambda b, j: (b, j, 0))],
        out_specs=pl.BlockSpec((SHELL_BB, LENGTH, DIM), lambda b, j: (b, j + 1, 0)),
        out_shape=jax.ShapeDtypeStruct((BATCH, LENGTH + SEQ, DIM), jnp.float32),
        compiler_params=pltpu.CompilerParams(
            dimension_semantics=("parallel", "arbitrary"),
        ),
    )(x_embed)


def _assemble_body(mean_ref, shell_ref, out_ref):
    del shell_ref
    out_ref[...] = mean_ref[...]


def _assemble(mean3, shell):
    return pl.pallas_call(
        _assemble_body,
        grid=(BATCH // ASM_BB,),
        in_specs=[
            pl.BlockSpec((ASM_BB, LENGTH, DIM), lambda b: (b, 0, 0)),
            pl.BlockSpec(memory_space=pltpu.MemorySpace.HBM),
        ],
        out_specs=pl.BlockSpec((ASM_BB, LENGTH, DIM), lambda b: (b, 0, 0)),
        out_shape=jax.ShapeDtypeStruct((BATCH, LENGTH + SEQ, DIM), jnp.float32),
        input_output_aliases={1: 0},
        compiler_params=pltpu.CompilerParams(
            dimension_semantics=("arbitrary",),
        ),
    )(mean3, shell)


def kernel(x_embed, x_key, prompt, prompt_key):
    shell = _shell(x_embed)
    idxp, w16, rs = _sim_topk(x_key, prompt_key)
    mean = _sc_combine(prompt, idxp.reshape(BATCH * 2 * TOPK), w16.reshape(BATCH * TOPK * 16))
    out = _assemble(mean, shell)
    return rs[0, 0], out


# speedup vs baseline: 25.7913x; 1.0461x over previous
import functools

import jax
import jax.numpy as jnp
from jax import lax
from jax.experimental import pallas as pl
from jax.experimental.pallas import tpu as pltpu
from jax.experimental.pallas import tpu_sc as plsc

POOL = 4096
LENGTH = 8
DIM = 1024
TOPK = 4
BATCH = 1024
SEQ = 64
KDIM = 2048

PBLK = 1024
NPBLK = POOL // PBLK

NCORES = 2
NSUB = 16
NWORK = NCORES * NSUB
BPW = BATCH // NWORK
IPW = BPW * TOPK
HALF = (LENGTH * DIM) // 2


def _sim_topk_body(x_ref, pk_ref, idx_out, w16_out, rs_out, vals_s, idxs_s, xinv_s):
    p = pl.program_id(0)

    @pl.when(p == 0)
    def _init():
        x = x_ref[...]
        xn = jnp.sqrt(jnp.sum(x * x, axis=1, keepdims=True))
        xinv_s[...] = 1.0 / jnp.maximum(xn, 1e-12)
        vals_s[...] = jnp.full((BATCH, TOPK), -jnp.inf, jnp.float32)
        idxs_s[...] = jnp.zeros((BATCH, TOPK), jnp.float32)

    pk = pk_ref[...]
    pkn = pk * (1.0 / jnp.maximum(jnp.sqrt(jnp.sum(pk * pk, axis=1, keepdims=True)), 1e-12))
    sim = lax.dot_general(x_ref[...], pkn, (((1,), (1,)), ((), ())),
                          preferred_element_type=jnp.float32)
    sim = sim * xinv_s[...]

    col = lax.broadcasted_iota(jnp.int32, (BATCH, PBLK), 1).astype(jnp.float32)
    bv, bi = [], []
    v = sim
    for _ in range(TOPK):
        m = jnp.max(v, axis=1, keepdims=True)
        ii = jnp.min(jnp.where(v == m, col, jnp.float32(PBLK)), axis=1, keepdims=True)
        bv.append(m)
        bi.append(ii + jnp.float32(p * PBLK))
        v = jnp.where(col == ii, -jnp.inf, v)
    bv = jnp.concatenate(bv, axis=1)
    bi = jnp.concatenate(bi, axis=1)

    cv = jnp.concatenate([vals_s[...], bv], axis=1)
    ci = jnp.concatenate([idxs_s[...], bi], axis=1)
    nv, ni = [], []
    for _ in range(TOPK):
        m = jnp.max(cv, axis=1, keepdims=True)
        ii = jnp.min(jnp.where(cv == m, ci, jnp.float32(2 ** 24)), axis=1, keepdims=True)
        nv.append(m)
        ni.append(ii)
        cv = jnp.where((cv == m) & (ci == ii), -jnp.inf, cv)
    vals = jnp.concatenate(nv, axis=1)
    idxs = jnp.concatenate(ni, axis=1)
    vals_s[...] = vals
    idxs_s[...] = idxs

    @pl.when(p == NPBLK - 1)
    def _fin():
        idxi = idxs.astype(jnp.int32)
        idx_out[...] = jnp.concatenate([idxi, idxi], axis=1)
        mx = jnp.max(vals, axis=1, keepdims=True)
        e = jnp.exp(vals - mx)
        w = e / jnp.sum(e, axis=1, keepdims=True) / TOPK
        w16_out[...] = jnp.broadcast_to(w[:, :, None], (BATCH, TOPK, 16))
        rs_out[...] = (jnp.sum(vals) / BATCH).reshape(1, 1)


def _sim_topk(x_key, prompt_key):
    return pl.pallas_call(
        _sim_topk_body,
        grid=(NPBLK,),
        in_specs=[
            pl.BlockSpec((BATCH, KDIM), lambda p: (0, 0)),
            pl.BlockSpec((PBLK, KDIM), lambda p: (p, 0)),
        ],
        out_specs=[
            pl.BlockSpec((BATCH, 2 * TOPK), lambda p: (0, 0)),
            pl.BlockSpec((BATCH, TOPK, 16), lambda p: (0, 0, 0)),
            pl.BlockSpec((1, 1), lambda p: (0, 0)),
        ],
        out_shape=[
            jax.ShapeDtypeStruct((BATCH, 2 * TOPK), jnp.int32),
            jax.ShapeDtypeStruct((BATCH, TOPK, 16), jnp.float32),
            jax.ShapeDtypeStruct((1, 1), jnp.float32),
        ],
        scratch_shapes=[
            pltpu.VMEM((BATCH, TOPK), jnp.float32),
            pltpu.VMEM((BATCH, TOPK), jnp.float32),
            pltpu.VMEM((BATCH, 1), jnp.float32),
        ],
        compiler_params=pltpu.CompilerParams(
            dimension_semantics=("arbitrary",),
        ),
    )(x_key, prompt_key)


def _sc_combine_body(tab_ref, idxp_ref, w_ref, out_ref,
                     idxp_v, w_v, buf_a, buf_b, buf_c, sem_a, sem_b, sem_c):
    cid = lax.axis_index("c")
    sid = lax.axis_index("s")
    wid = sid * NCORES + cid
    base_i = wid * IPW

    pltpu.sync_copy(idxp_ref.at[pl.ds(base_i * 2, IPW * 2)], idxp_v)
    pltpu.sync_copy(w_ref.at[pl.ds(base_i * 16, IPW * 16)], w_v)

    bufs = (buf_a, buf_b, buf_c)
    sems = (sem_a, sem_b, sem_c)

    def start(j, buf, sem):
        pltpu.async_copy(tab_ref.at[idxp_v.at[pl.ds(j * 8, TOPK)]], buf, sem)

    def drain(buf, sem):
        pltpu.make_async_copy(tab_ref.at[pl.ds(0, TOPK)], buf, sem).wait()

    def combine(j, buf):
        w0 = w_v[pl.ds((j * TOPK + 0) * 16, 16)]
        w1 = w_v[pl.ds((j * TOPK + 1) * 16, 16)]
        w2 = w_v[pl.ds((j * TOPK + 2) * 16, 16)]
        w3 = w_v[pl.ds((j * TOPK + 3) * 16, 16)]

        def chunk(ci2, _):
            for u in range(2):
                off = (ci2 * 2 + u) * 16
                for l in range(LENGTH):
                    acc = w0 * buf[0, l, pl.ds(off, 16)]
                    acc = acc + w1 * buf[1, l, pl.ds(off, 16)]
                    acc = acc + w2 * buf[2, l, pl.ds(off, 16)]
                    acc = acc + w3 * buf[3, l, pl.ds(off, 16)]
                    buf[0, l, pl.ds(off, 16)] = acc
            return 0
        lax.fori_loop(0, DIM // 32, chunk, 0)
        pltpu.sync_copy(buf.at[0], out_ref.at[wid * BPW + j])

    for s in range(3):
        start(s, bufs[s], sems[s])

    def ring(t, _):
        for s in range(3):
            j = 3 * t + s

            @pl.when(j < BPW)
            def _():
                drain(bufs[s], sems[s])
                combine(j, bufs[s])

            @pl.when(j + 3 < BPW)
            def _():
                start(j + 3, bufs[s], sems[s])
        return 0

    lax.fori_loop(0, (BPW + 2) // 3, ring, 0)


def _sc_combine(prompt, idx_pad, w_flat):
    mesh = plsc.VectorSubcoreMesh(core_axis_name="c", subcore_axis_name="s")
    fn = functools.partial(
        pl.kernel,
        out_type=jax.ShapeDtypeStruct((BATCH, LENGTH, DIM), jnp.float32),
        mesh=mesh,
        scratch_types=[
            pltpu.VMEM((2 * IPW,), jnp.int32),
            pltpu.VMEM((IPW * 16,), jnp.float32),
            pltpu.VMEM((TOPK, LENGTH, DIM), jnp.float32),
            pltpu.VMEM((TOPK, LENGTH, DIM), jnp.float32),
            pltpu.VMEM((TOPK, LENGTH, DIM), jnp.float32),
            pltpu.SemaphoreType.DMA,
            pltpu.SemaphoreType.DMA,
            pltpu.SemaphoreType.DMA,
        ],
        compiler_params=pltpu.CompilerParams(use_tc_tiling_on_sc=True),
    )(_sc_combine_body)
    return fn(prompt, idx_pad, w_flat)


SHELL_BB = 256
ASM_BB = 128


def _shell_body(x_ref, out_ref):
    out_ref[...] = x_ref[...]


def _shell(x_embed):
    return pl.pallas_call(
        _shell_body,
        grid=(BATCH // SHELL_BB, SEQ // LENGTH),
        in_specs=[pl.BlockSpec((SHELL_BB, LENGTH, DIM), lambda b, j: (b, j, 0))],
        out_specs=pl.BlockSpec((SHELL_BB, LENGTH, DIM), lambda b, j: (b, j + 1, 0)),
        out_shape=jax.ShapeDtypeStruct((BATCH, LENGTH + SEQ, DIM), jnp.float32),
        compiler_params=pltpu.CompilerParams(
            dimension_semantics=("parallel", "arbitrary"),
        ),
    )(x_embed)


def _assemble_body(mean_ref, shell_ref, out_ref):
    del shell_ref
    out_ref[...] = mean_ref[...]


def _assemble(mean3, shell):
    return pl.pallas_call(
        _assemble_body,
        grid=(BATCH // ASM_BB,),
        in_specs=[
            pl.BlockSpec((ASM_BB, LENGTH, DIM), lambda b: (b, 0, 0)),
            pl.BlockSpec(memory_space=pltpu.MemorySpace.HBM),
        ],
        out_specs=pl.BlockSpec((ASM_BB, LENGTH, DIM), lambda b: (b, 0, 0)),
        out_shape=jax.ShapeDtypeStruct((BATCH, LENGTH + SEQ, DIM), jnp.float32),
        input_output_aliases={1: 0},
        compiler_params=pltpu.CompilerParams(
            dimension_semantics=("arbitrary",),
        ),
    )(mean3, shell)


def kernel(x_embed, x_key, prompt, prompt_key):
    shell = _shell(x_embed)
    idxp, w16, rs = _sim_topk(x_key, prompt_key)
    mean = _sc_combine(prompt, idxp.reshape(BATCH * 2 * TOPK), w16.reshape(BATCH * TOPK * 16))
    out = _assemble(mean, shell)
    return rs[0, 0], out


# speedup vs baseline: 26.2096x; 1.0162x over previous
import functools

import jax
import jax.numpy as jnp
from jax import lax
from jax.experimental import pallas as pl
from jax.experimental.pallas import tpu as pltpu
from jax.experimental.pallas import tpu_sc as plsc

POOL = 4096
LENGTH = 8
DIM = 1024
TOPK = 4
BATCH = 1024
SEQ = 64
KDIM = 2048

PBLK = 2048
NPBLK = POOL // PBLK

NCORES = 2
NSUB = 16
NWORK = NCORES * NSUB
BPW = BATCH // NWORK
IPW = BPW * TOPK
HALF = (LENGTH * DIM) // 2


def _sim_topk_body(x_ref, pk_ref, idx_out, w16_out, rs_out, vals_s, idxs_s, xinv_s):
    p = pl.program_id(0)

    @pl.when(p == 0)
    def _init():
        x = x_ref[...]
        xn = jnp.sqrt(jnp.sum(x * x, axis=1, keepdims=True))
        xinv_s[...] = 1.0 / jnp.maximum(xn, 1e-12)
        vals_s[...] = jnp.full((BATCH, TOPK), -jnp.inf, jnp.float32)
        idxs_s[...] = jnp.zeros((BATCH, TOPK), jnp.float32)

    pk = pk_ref[...]
    pkn = pk * (1.0 / jnp.maximum(jnp.sqrt(jnp.sum(pk * pk, axis=1, keepdims=True)), 1e-12))
    sim = lax.dot_general(x_ref[...], pkn, (((1,), (1,)), ((), ())),
                          preferred_element_type=jnp.float32)
    sim = sim * xinv_s[...]

    col = lax.broadcasted_iota(jnp.int32, (BATCH, PBLK), 1).astype(jnp.float32)
    bv, bi = [], []
    v = sim
    for _ in range(TOPK):
        m = jnp.max(v, axis=1, keepdims=True)
        ii = jnp.min(jnp.where(v == m, col, jnp.float32(PBLK)), axis=1, keepdims=True)
        bv.append(m)
        bi.append(ii + jnp.float32(p * PBLK))
        v = jnp.where(col == ii, -jnp.inf, v)
    bv = jnp.concatenate(bv, axis=1)
    bi = jnp.concatenate(bi, axis=1)

    cv = jnp.concatenate([vals_s[...], bv], axis=1)
    ci = jnp.concatenate([idxs_s[...], bi], axis=1)
    nv, ni = [], []
    for _ in range(TOPK):
        m = jnp.max(cv, axis=1, keepdims=True)
        ii = jnp.min(jnp.where(cv == m, ci, jnp.float32(2 ** 24)), axis=1, keepdims=True)
        nv.append(m)
        ni.append(ii)
        cv = jnp.where((cv == m) & (ci == ii), -jnp.inf, cv)
    vals = jnp.concatenate(nv, axis=1)
    idxs = jnp.concatenate(ni, axis=1)
    vals_s[...] = vals
    idxs_s[...] = idxs

    @pl.when(p == NPBLK - 1)
    def _fin():
        idxi = idxs.astype(jnp.int32)
        idx_out[...] = jnp.concatenate([idxi, idxi], axis=1)
        mx = jnp.max(vals, axis=1, keepdims=True)
        e = jnp.exp(vals - mx)
        w = e / jnp.sum(e, axis=1, keepdims=True) / TOPK
        w16_out[...] = jnp.broadcast_to(w[:, :, None], (BATCH, TOPK, 16))
        rs_out[...] = (jnp.sum(vals) / BATCH).reshape(1, 1)


def _sim_topk(x_key, prompt_key):
    return pl.pallas_call(
        _sim_topk_body,
        grid=(NPBLK,),
        in_specs=[
            pl.BlockSpec((BATCH, KDIM), lambda p: (0, 0)),
            pl.BlockSpec((PBLK, KDIM), lambda p: (p, 0)),
        ],
        out_specs=[
            pl.BlockSpec((BATCH, 2 * TOPK), lambda p: (0, 0)),
            pl.BlockSpec((BATCH, TOPK, 16), lambda p: (0, 0, 0)),
            pl.BlockSpec((1, 1), lambda p: (0, 0)),
        ],
        out_shape=[
            jax.ShapeDtypeStruct((BATCH, 2 * TOPK), jnp.int32),
            jax.ShapeDtypeStruct((BATCH, TOPK, 16), jnp.float32),
            jax.ShapeDtypeStruct((1, 1), jnp.float32),
        ],
        scratch_shapes=[
            pltpu.VMEM((BATCH, TOPK), jnp.float32),
            pltpu.VMEM((BATCH, TOPK), jnp.float32),
            pltpu.VMEM((BATCH, 1), jnp.float32),
        ],
        compiler_params=pltpu.CompilerParams(
            dimension_semantics=("arbitrary",),
        ),
    )(x_key, prompt_key)


def _sc_combine_body(tab_ref, idxp_ref, w_ref, out_ref,
                     idxp_v, w_v, buf_a, buf_b, buf_c, sem_a, sem_b, sem_c):
    cid = lax.axis_index("c")
    sid = lax.axis_index("s")
    wid = sid * NCORES + cid
    base_i = wid * IPW

    pltpu.sync_copy(idxp_ref.at[pl.ds(base_i * 2, IPW * 2)], idxp_v)
    pltpu.sync_copy(w_ref.at[pl.ds(base_i * 16, IPW * 16)], w_v)

    bufs = (buf_a, buf_b, buf_c)
    sems = (sem_a, sem_b, sem_c)

    def start(j, buf, sem):
        pltpu.async_copy(tab_ref.at[idxp_v.at[pl.ds(j * 8, TOPK)]], buf, sem)

    def drain(buf, sem):
        pltpu.make_async_copy(tab_ref.at[pl.ds(0, TOPK)], buf, sem).wait()

    def combine(j, buf):
        w0 = w_v[pl.ds((j * TOPK + 0) * 16, 16)]
        w1 = w_v[pl.ds((j * TOPK + 1) * 16, 16)]
        w2 = w_v[pl.ds((j * TOPK + 2) * 16, 16)]
        w3 = w_v[pl.ds((j * TOPK + 3) * 16, 16)]

        def chunk(ci2, _):
            for u in range(2):
                off = (ci2 * 2 + u) * 16
                for l in range(LENGTH):
                    acc = w0 * buf[0, l, pl.ds(off, 16)]
                    acc = acc + w1 * buf[1, l, pl.ds(off, 16)]
                    acc = acc + w2 * buf[2, l, pl.ds(off, 16)]
                    acc = acc + w3 * buf[3, l, pl.ds(off, 16)]
                    buf[0, l, pl.ds(off, 16)] = acc
            return 0
        lax.fori_loop(0, DIM // 32, chunk, 0)
        pltpu.sync_copy(buf.at[0], out_ref.at[wid * BPW + j])

    for s in range(3):
        start(s, bufs[s], sems[s])

    def ring(t, _):
        for s in range(3):
            j = 3 * t + s

            @pl.when(j < BPW)
            def _():
                drain(bufs[s], sems[s])
                combine(j, bufs[s])

            @pl.when(j + 3 < BPW)
            def _():
                start(j + 3, bufs[s], sems[s])
        return 0

    lax.fori_loop(0, (BPW + 2) // 3, ring, 0)


def _sc_combine(prompt, idx_pad, w_flat):
    mesh = plsc.VectorSubcoreMesh(core_axis_name="c", subcore_axis_name="s")
    fn = functools.partial(
        pl.kernel,
        out_type=jax.ShapeDtypeStruct((BATCH, LENGTH, DIM), jnp.float32),
        mesh=mesh,
        scratch_types=[
            pltpu.VMEM((2 * IPW,), jnp.int32),
            pltpu.VMEM((IPW * 16,), jnp.float32),
            pltpu.VMEM((TOPK, LENGTH, DIM), jnp.float32),
            pltpu.VMEM((TOPK, LENGTH, DIM), jnp.float32),
            pltpu.VMEM((TOPK, LENGTH, DIM), jnp.float32),
            pltpu.SemaphoreType.DMA,
            pltpu.SemaphoreType.DMA,
            pltpu.SemaphoreType.DMA,
        ],
        compiler_params=pltpu.CompilerParams(use_tc_tiling_on_sc=True),
    )(_sc_combine_body)
    return fn(prompt, idx_pad, w_flat)


SHELL_BB = 256
ASM_BB = 128


def _shell_body(x_ref, out_ref):
    out_ref[...] = x_ref[...]


def _shell(x_embed):
    return pl.pallas_call(
        _shell_body,
        grid=(BATCH // SHELL_BB, SEQ // LENGTH),
        in_specs=[pl.BlockSpec((SHELL_BB, LENGTH, DIM), lambda b, j: (b, j, 0))],
        out_specs=pl.BlockSpec((SHELL_BB, LENGTH, DIM), lambda b, j: (b, j + 1, 0)),
        out_shape=jax.ShapeDtypeStruct((BATCH, LENGTH + SEQ, DIM), jnp.float32),
        compiler_params=pltpu.CompilerParams(
            dimension_semantics=("parallel", "arbitrary"),
        ),
    )(x_embed)


def _assemble_body(mean_ref, shell_ref, out_ref):
    del shell_ref
    out_ref[...] = mean_ref[...]


def _assemble(mean3, shell):
    return pl.pallas_call(
        _assemble_body,
        grid=(BATCH // ASM_BB,),
        in_specs=[
            pl.BlockSpec((ASM_BB, LENGTH, DIM), lambda b: (b, 0, 0)),
            pl.BlockSpec(memory_space=pltpu.MemorySpace.HBM),
        ],
        out_specs=pl.BlockSpec((ASM_BB, LENGTH, DIM), lambda b: (b, 0, 0)),
        out_shape=jax.ShapeDtypeStruct((BATCH, LENGTH + SEQ, DIM), jnp.float32),
        input_output_aliases={1: 0},
        compiler_params=pltpu.CompilerParams(
            dimension_semantics=("arbitrary",),
        ),
    )(mean3, shell)


def kernel(x_embed, x_key, prompt, prompt_key):
    shell = _shell(x_embed)
    idxp, w16, rs = _sim_topk(x_key, prompt_key)
    mean = _sc_combine(prompt, idxp.reshape(BATCH * 2 * TOPK), w16.reshape(BATCH * TOPK * 16))
    out = _assemble(mean, shell)
    return rs[0, 0], out
